# baseline (speedup 1.0000x reference)
import functools

import jax
import jax.numpy as jnp
from jax import lax
from jax.experimental import pallas as pl
from jax.experimental.pallas import tpu as pltpu
from jax.experimental.pallas import tpu_sc as plsc

N_NODES = 10000
F = 128
E_EDGES = 320000
L_LAYERS = 4
ALPHA = 0.1

NC = 2
NS = 16
NW = NC * NS
CH = 128
CPT = 80
EPT = CPT * CH
EP = NW * EPT
GRP = 8
NIT = CPT // GRP
ROWS_PAD = 10240
RPT = ROWS_PAD // NS

BLK = 400
NBLK = N_NODES // BLK

_mesh = plsc.VectorSubcoreMesh(core_axis_name="c", subcore_axis_name="s")


def _zero_fill(ref, nvec):
    z = jnp.zeros((16,), jnp.float32)

    def body(i, _):
        ref[i // 8, pl.ds((i % 8) * 16, 16)] = z
        return 0

    lax.fori_loop(0, nvec, body, 0)


@functools.partial(
    pl.kernel,
    out_type=jax.ShapeDtypeStruct((NC, ROWS_PAD), jnp.float32),
    mesh=_mesh,
    scratch_types=[
        pltpu.VMEM_SHARED((ROWS_PAD,), jnp.float32),
        pltpu.VMEM((CH,), jnp.float32),
        pltpu.VMEM((RPT,), jnp.float32),
        pltpu.VMEM((CPT, CH), jnp.int32),
    ],
)
def _sc_degree(dst_hbm, out_hbm, acc, ones_v, zeros_v, dst_all):
    cid = lax.axis_index("c")
    sid = lax.axis_index("s")
    wid = cid * NS + sid

    pltpu.sync_copy(dst_hbm.at[wid], dst_all)

    one = jnp.ones((16,), jnp.float32)
    zero = jnp.zeros((16,), jnp.float32)

    def fill(i, _):
        ones_v[pl.ds(i * 16, 16)] = one
        return 0

    lax.fori_loop(0, CH // 16, fill, 0)

    def zfill(i, _):
        zeros_v[pl.ds(i * 16, 16)] = zero
        return 0

    lax.fori_loop(0, RPT // 16, zfill, 0)

    rbase = sid * RPT
    pltpu.sync_copy(zeros_v, acc.at[pl.ds(rbase, RPT)])
    plsc.subcore_barrier()

    def body(i, _):
        pltpu.sync_copy(ones_v, acc.at[dst_all.at[i]], add=True)
        return 0

    lax.fori_loop(0, CPT, body, 0)
    plsc.subcore_barrier()
    pltpu.sync_copy(acc.at[pl.ds(rbase, RPT)], out_hbm.at[cid, pl.ds(rbase, RPT)])


@functools.partial(
    pl.kernel,
    out_type=jax.ShapeDtypeStruct((NC, ROWS_PAD, F), jnp.float32),
    mesh=_mesh,
    scratch_types=[
        pltpu.VMEM_SHARED((ROWS_PAD, F), jnp.float32),
        pltpu.VMEM((64, F), jnp.float32),
        pltpu.VMEM((4, CH), jnp.int32),
        pltpu.VMEM((4, CH), jnp.int32),
        pltpu.VMEM((4, CH), jnp.int32),
        pltpu.VMEM((4, CH), jnp.int32),
        pltpu.VMEM((CH, F), jnp.float32),
        pltpu.VMEM((CH, F), jnp.float32),
        pltpu.SemaphoreType.DMA,
        pltpu.SemaphoreType.DMA,
        pltpu.SemaphoreType.DMA,
        pltpu.SemaphoreType.DMA,
        pltpu.SemaphoreType.DMA,
        pltpu.SemaphoreType.DMA,
    ],
)
def _sc_propagate(hns_hbm, src_hbm, dst_hbm, out_hbm, acc, zbuf, src0, dst0,
                  src1, dst1, rows_a, rows_b, gsa, gsb, ssa, ssb, is0, is1):
    cid = lax.axis_index("c")
    sid = lax.axis_index("s")
    wid = cid * NS + sid
    rbase = sid * RPT

    def load_idx(cbase, sbuf, dbuf, sem):
        pltpu.async_copy(src_hbm.at[wid, pl.ds(cbase, 4)], sbuf, sem)
        pltpu.async_copy(dst_hbm.at[wid, pl.ds(cbase, 4)], dbuf, sem)

    def wait_idx(sbuf, dbuf, sem):
        pltpu.make_async_copy(src_hbm.at[wid, pl.ds(0, 4)], sbuf, sem).wait()
        pltpu.make_async_copy(dst_hbm.at[wid, pl.ds(0, 4)], dbuf, sem).wait()

    def fire_g(sbuf, j, rbuf, sem):
        pltpu.async_copy(hns_hbm.at[sbuf.at[j]], rbuf, sem)

    def wait_g(rbuf, sem):
        pltpu.make_async_copy(hns_hbm.at[src0.at[0]], rbuf, sem).wait()

    def fire_s(rbuf, dbuf, j, sem):
        pltpu.async_copy(rbuf, acc.at[dbuf.at[j]], sem, add=True)

    def wait_s(rbuf, sem):
        pltpu.make_async_copy(rbuf, acc.at[dst0.at[0]], sem).wait()

    load_idx(0, src0, dst0, is0)
    _zero_fill(zbuf, 64 * 8)
    for r in range(0, RPT, 64):
        pltpu.sync_copy(zbuf, acc.at[pl.ds(rbase + r, 64)])
    wait_idx(src0, dst0, is0)
    fire_g(src0, 0, rows_a, gsa)
    fire_g(src0, 1, rows_b, gsb)
    plsc.subcore_barrier()

    def group(k, _):
        base = k * GRP
        load_idx(base + 4, src1, dst1, is1)
        wait_g(rows_a, gsa)
        fire_s(rows_a, dst0, 0, ssa)
        wait_s(rows_a, ssa)
        fire_g(src0, 2, rows_a, gsa)
        wait_g(rows_b, gsb)
        fire_s(rows_b, dst0, 1, ssb)
        wait_s(rows_b, ssb)
        fire_g(src0, 3, rows_b, gsb)
        wait_g(rows_a, gsa)
        fire_s(rows_a, dst0, 2, ssa)
        wait_s(rows_a, ssa)
        wait_idx(src1, dst1, is1)
        fire_g(src1, 0, rows_a, gsa)
        wait_g(rows_b, gsb)
        fire_s(rows_b, dst0, 3, ssb)
        wait_s(rows_b, ssb)
        fire_g(src1, 1, rows_b, gsb)

        @pl.when(k < NIT - 1)
        def _():
            load_idx(base + GRP, src0, dst0, is0)

        wait_g(rows_a, gsa)
        fire_s(rows_a, dst1, 0, ssa)
        wait_s(rows_a, ssa)
        fire_g(src1, 2, rows_a, gsa)
        wait_g(rows_b, gsb)
        fire_s(rows_b, dst1, 1, ssb)
        wait_s(rows_b, ssb)
        fire_g(src1, 3, rows_b, gsb)
        wait_g(rows_a, gsa)
        fire_s(rows_a, dst1, 2, ssa)
        wait_s(rows_a, ssa)

        @pl.when(k < NIT - 1)
        def _():
            wait_idx(src0, dst0, is0)
            fire_g(src0, 0, rows_a, gsa)

        wait_g(rows_b, gsb)
        fire_s(rows_b, dst1, 3, ssb)
        wait_s(rows_b, ssb)

        @pl.when(k < NIT - 1)
        def _():
            fire_g(src0, 1, rows_b, gsb)

        return 0

    lax.fori_loop(0, NIT, group, 0)
    plsc.subcore_barrier()
    pltpu.sync_copy(acc.at[pl.ds(rbase, RPT)],
                    out_hbm.at[cid, pl.ds(rbase, RPT)])


def _dinv_body(d0_ref, d1_ref, o_ref):
    deg = d0_ref[...] + d1_ref[...] + 1.0
    o_ref[...] = lax.rsqrt(deg)


def _prelude_body(x_ref, w_ref, b_ref, h_ref, s_ref, q_ref):
    h = jnp.dot(x_ref[...], w_ref[...], preferred_element_type=jnp.float32)
    h = jnp.maximum(h + b_ref[...], 0.0)
    h_ref[...] = h

    @pl.when(pl.program_id(0) == 0)
    def _():
        s_ref[...] = jnp.zeros_like(s_ref)
        q_ref[...] = jnp.zeros_like(q_ref)

    s_ref[...] += jnp.sum(h, axis=0, keepdims=True)
    q_ref[...] += jnp.sum(h * h, axis=0, keepdims=True)


def _norm_body(h_ref, s_ref, q_ref, g_ref, b_ref, dinv_ref, hn_ref, hns_ref):
    inv_n = 1.0 / N_NODES
    mean = s_ref[...] * inv_n
    var = q_ref[...] * inv_n - mean * mean
    scale = lax.rsqrt(var + 1e-5) * g_ref[...]
    hn = (h_ref[...] - mean) * scale + b_ref[...]
    hn_ref[...] = hn
    hns_ref[...] = hn * dinv_ref[...]


def _combine_body(s0_ref, s1_ref, hn_ref, hns_ref, h0_ref, dinv_ref, wc_ref,
                  h_ref, s_ref, q_ref):
    ax = (s0_ref[...] + s1_ref[...] + hns_ref[...]) * dinv_ref[...]
    t = (1.0 - ALPHA) * ax + ALPHA * h0_ref[...]
    u = jnp.dot(t, wc_ref[...], preferred_element_type=jnp.float32)
    h = jnp.maximum(u, 0.0) + hn_ref[...]
    h_ref[...] = h

    @pl.when(pl.program_id(0) == 0)
    def _():
        s_ref[...] = jnp.zeros_like(s_ref)
        q_ref[...] = jnp.zeros_like(q_ref)

    s_ref[...] += jnp.sum(h, axis=0, keepdims=True)
    q_ref[...] += jnp.sum(h * h, axis=0, keepdims=True)


def _final_body(h_ref, w_ref, b_ref, o_ref):
    logits = jnp.dot(h_ref[...], w_ref[...],
                     preferred_element_type=jnp.float32) + b_ref[...]
    m = jnp.max(logits, axis=-1, keepdims=True)
    s = logits - m
    lse = jnp.log(jnp.sum(jnp.exp(s), axis=-1, keepdims=True))
    o_ref[...] = s - lse


def _row_spec(cols):
    return pl.BlockSpec((BLK, cols), lambda i: (i, 0))


def _bcast_spec(rows, cols):
    return pl.BlockSpec((rows, cols), lambda i: (0, 0))


_STATS_OUT = [
    jax.ShapeDtypeStruct((1, F), jnp.float32),
    jax.ShapeDtypeStruct((1, F), jnp.float32),
]

_dinv = pl.pallas_call(
    _dinv_body,
    grid=(),
    out_shape=jax.ShapeDtypeStruct((ROWS_PAD // F, F), jnp.float32),
)

_prelude = pl.pallas_call(
    _prelude_body,
    grid=(NBLK,),
    in_specs=[_row_spec(F), _bcast_spec(F, F), _bcast_spec(1, F)],
    out_specs=[_row_spec(F), _bcast_spec(1, F), _bcast_spec(1, F)],
    out_shape=[jax.ShapeDtypeStruct((N_NODES, F), jnp.float32)] + _STATS_OUT,
)

_norm = pl.pallas_call(
    _norm_body,
    grid=(NBLK,),
    in_specs=[_row_spec(F), _bcast_spec(1, F), _bcast_spec(1, F),
              _bcast_spec(1, F), _bcast_spec(1, F), _row_spec(1)],
    out_specs=[_row_spec(F), _row_spec(F)],
    out_shape=[jax.ShapeDtypeStruct((N_NODES, F), jnp.float32),
               jax.ShapeDtypeStruct((N_NODES, F), jnp.float32)],
)

_combine = pl.pallas_call(
    _combine_body,
    grid=(NBLK,),
    in_specs=[_row_spec(F), _row_spec(F), _row_spec(F), _row_spec(F),
              _row_spec(F), _row_spec(1), _bcast_spec(F, F)],
    out_specs=[_row_spec(F), _bcast_spec(1, F), _bcast_spec(1, F)],
    out_shape=[jax.ShapeDtypeStruct((N_NODES, F), jnp.float32)] + _STATS_OUT,
)

_final = pl.pallas_call(
    _final_body,
    grid=(NBLK,),
    in_specs=[_row_spec(F), _bcast_spec(F, 40), _bcast_spec(1, 40)],
    out_specs=_row_spec(40),
    out_shape=jax.ShapeDtypeStruct((N_NODES, 40), jnp.float32),
)


def kernel(x, adj_t, W1, b1, gammas, bn_betas, Wc, W2, b2):
    pad = EP - E_EDGES
    src_p = jnp.concatenate([adj_t[0], jnp.zeros((pad,), jnp.int32)])
    src_p = src_p.reshape(NW, CPT, CH)
    dst_p = jnp.concatenate([adj_t[1], jnp.full((pad,), N_NODES, jnp.int32)])
    dst_p = dst_p.reshape(NW, CPT, CH)

    deg2 = _sc_degree(dst_p)
    dinv2d = _dinv(deg2[0].reshape(ROWS_PAD // F, F),
                   deg2[1].reshape(ROWS_PAD // F, F))
    dinv_col = dinv2d.reshape(ROWS_PAD, 1)[:N_NODES]

    h, ssum, ssq = _prelude(x, W1, b1.reshape(1, F))
    h0 = h
    for l in range(L_LAYERS):
        hn, hns = _norm(h, ssum, ssq, gammas[l].reshape(1, F),
                        bn_betas[l].reshape(1, F), dinv_col)
        s2 = _sc_propagate(hns, src_p, dst_p)
        h, ssum, ssq = _combine(s2[0, :N_NODES], s2[1, :N_NODES], hn, hns,
                                h0, dinv_col, Wc[l])
    return _final(h, W2, b2.reshape(1, 40))


# speedup vs baseline: 3.1869x; 3.1869x over previous
import functools

import jax
import jax.numpy as jnp
from jax import lax
from jax.experimental import pallas as pl
from jax.experimental.pallas import tpu as pltpu
from jax.experimental.pallas import tpu_sc as plsc

N_NODES = 10000
F = 128
E_EDGES = 320000
L_LAYERS = 4
ALPHA = 0.1

NC = 2
NS = 16
NW = NC * NS
CH = 128
CPT = 80
EPT = CPT * CH
EP = NW * EPT
GRP = 8
NIT = CPT // GRP
ROWS_PAD = 10240
RPT = ROWS_PAD // NS

BLK = 400
NBLK = N_NODES // BLK

_mesh = plsc.VectorSubcoreMesh(core_axis_name="c", subcore_axis_name="s")


def _zero_fill(ref, nvec):
    z = jnp.zeros((16,), jnp.float32)

    def body(i, _):
        ref[i // 8, pl.ds((i % 8) * 16, 16)] = z
        return 0

    lax.fori_loop(0, nvec, body, 0)


@functools.partial(
    pl.kernel,
    out_type=jax.ShapeDtypeStruct((NC, ROWS_PAD), jnp.float32),
    mesh=_mesh,
    scratch_types=[
        pltpu.VMEM_SHARED((ROWS_PAD,), jnp.float32),
        pltpu.VMEM((CH,), jnp.float32),
        pltpu.VMEM((RPT,), jnp.float32),
        pltpu.VMEM((CPT, CH), jnp.int32),
    ],
)
def _sc_degree(dst_hbm, out_hbm, acc, ones_v, zeros_v, dst_all):
    cid = lax.axis_index("c")
    sid = lax.axis_index("s")
    wid = cid * NS + sid

    pltpu.sync_copy(dst_hbm.at[wid], dst_all)

    one = jnp.ones((16,), jnp.float32)
    zero = jnp.zeros((16,), jnp.float32)

    def fill(i, _):
        ones_v[pl.ds(i * 16, 16)] = one
        return 0

    lax.fori_loop(0, CH // 16, fill, 0)

    def zfill(i, _):
        zeros_v[pl.ds(i * 16, 16)] = zero
        return 0

    lax.fori_loop(0, RPT // 16, zfill, 0)

    rbase = sid * RPT
    pltpu.sync_copy(zeros_v, acc.at[pl.ds(rbase, RPT)])
    plsc.subcore_barrier()

    def body(i, _):
        pltpu.sync_copy(ones_v, acc.at[dst_all.at[i]], add=True)
        return 0

    lax.fori_loop(0, CPT, body, 0)
    plsc.subcore_barrier()
    pltpu.sync_copy(acc.at[pl.ds(rbase, RPT)], out_hbm.at[cid, pl.ds(rbase, RPT)])


@functools.partial(
    pl.kernel,
    out_type=jax.ShapeDtypeStruct((NC, ROWS_PAD, F), jnp.float32),
    mesh=_mesh,
    scratch_types=[
        pltpu.VMEM_SHARED((ROWS_PAD, F), jnp.float32),
        pltpu.VMEM((64, F), jnp.float32),
        pltpu.VMEM((4, CH), jnp.int32),
        pltpu.VMEM((4, CH), jnp.int32),
        pltpu.VMEM((4, CH), jnp.int32),
        pltpu.VMEM((4, CH), jnp.int32),
        pltpu.VMEM((CH, F), jnp.float32),
        pltpu.VMEM((CH, F), jnp.float32),
        pltpu.SemaphoreType.DMA,
        pltpu.SemaphoreType.DMA,
        pltpu.SemaphoreType.DMA,
        pltpu.SemaphoreType.DMA,
        pltpu.SemaphoreType.DMA,
        pltpu.SemaphoreType.DMA,
    ],
)
def _sc_propagate(hns_hbm, src_hbm, dst_hbm, out_hbm, acc, zbuf, src0, dst0,
                  src1, dst1, rows_a, rows_b, gsa, gsb, ssa, ssb, is0, is1):
    cid = lax.axis_index("c")
    sid = lax.axis_index("s")
    wid = cid * NS + sid
    rbase = sid * RPT

    def load_idx(cbase, sbuf, dbuf, sem):
        pltpu.async_copy(src_hbm.at[wid, pl.ds(cbase, 4)], sbuf, sem)
        pltpu.async_copy(dst_hbm.at[wid, pl.ds(cbase, 4)], dbuf, sem)

    def wait_idx(sbuf, dbuf, sem):
        pltpu.make_async_copy(src_hbm.at[wid, pl.ds(0, 4)], sbuf, sem).wait()
        pltpu.make_async_copy(dst_hbm.at[wid, pl.ds(0, 4)], dbuf, sem).wait()

    def fire_g(sbuf, j, rbuf, sem):
        pltpu.async_copy(hns_hbm.at[sbuf.at[j]], rbuf, sem)

    def wait_g(rbuf, sem):
        pltpu.make_async_copy(hns_hbm.at[src0.at[0]], rbuf, sem).wait()

    def fire_s(rbuf, dbuf, j, sem):
        pltpu.async_copy(rbuf, acc.at[dbuf.at[j]], sem, add=True)

    def wait_s(rbuf, sem):
        pltpu.make_async_copy(rbuf, acc.at[dst0.at[0]], sem).wait()

    load_idx(0, src0, dst0, is0)
    _zero_fill(zbuf, 64 * 8)
    for r in range(0, RPT, 64):
        pltpu.sync_copy(zbuf, acc.at[pl.ds(rbase + r, 64)])
    wait_idx(src0, dst0, is0)
    fire_g(src0, 0, rows_a, gsa)
    fire_g(src0, 1, rows_b, gsb)
    plsc.subcore_barrier()

    def group(k, _):
        base = k * GRP
        load_idx(base + 4, src1, dst1, is1)
        wait_g(rows_a, gsa)
        fire_s(rows_a, dst0, 0, ssa)
        wait_s(rows_a, ssa)
        fire_g(src0, 2, rows_a, gsa)
        wait_g(rows_b, gsb)
        fire_s(rows_b, dst0, 1, ssb)
        wait_s(rows_b, ssb)
        fire_g(src0, 3, rows_b, gsb)
        wait_g(rows_a, gsa)
        fire_s(rows_a, dst0, 2, ssa)
        wait_s(rows_a, ssa)
        wait_idx(src1, dst1, is1)
        fire_g(src1, 0, rows_a, gsa)
        wait_g(rows_b, gsb)
        fire_s(rows_b, dst0, 3, ssb)
        wait_s(rows_b, ssb)
        fire_g(src1, 1, rows_b, gsb)

        @pl.when(k < NIT - 1)
        def _():
            load_idx(base + GRP, src0, dst0, is0)

        wait_g(rows_a, gsa)
        fire_s(rows_a, dst1, 0, ssa)
        wait_s(rows_a, ssa)
        fire_g(src1, 2, rows_a, gsa)
        wait_g(rows_b, gsb)
        fire_s(rows_b, dst1, 1, ssb)
        wait_s(rows_b, ssb)
        fire_g(src1, 3, rows_b, gsb)
        wait_g(rows_a, gsa)
        fire_s(rows_a, dst1, 2, ssa)
        wait_s(rows_a, ssa)

        @pl.when(k < NIT - 1)
        def _():
            wait_idx(src0, dst0, is0)
            fire_g(src0, 0, rows_a, gsa)

        wait_g(rows_b, gsb)
        fire_s(rows_b, dst1, 3, ssb)
        wait_s(rows_b, ssb)

        @pl.when(k < NIT - 1)
        def _():
            fire_g(src0, 1, rows_b, gsb)

        return 0

    lax.fori_loop(0, NIT, group, 0)
    plsc.subcore_barrier()
    pltpu.sync_copy(acc.at[pl.ds(rbase, RPT)],
                    out_hbm.at[cid, pl.ds(rbase, RPT)])


def _dinv_body(d0_ref, d1_ref, o_ref):
    deg = d0_ref[...] + d1_ref[...] + 1.0
    o_ref[...] = lax.rsqrt(deg)


def _prelude_body(x_ref, w_ref, b_ref, h_ref, s_ref, q_ref):
    h = jnp.dot(x_ref[...], w_ref[...], preferred_element_type=jnp.float32)
    h = jnp.maximum(h + b_ref[...], 0.0)
    h_ref[...] = h

    @pl.when(pl.program_id(0) == 0)
    def _():
        s_ref[...] = jnp.zeros_like(s_ref)
        q_ref[...] = jnp.zeros_like(q_ref)

    s_ref[...] += jnp.sum(h, axis=0, keepdims=True)
    q_ref[...] += jnp.sum(h * h, axis=0, keepdims=True)


def _norm_body(h_ref, s_ref, q_ref, g_ref, b_ref, dinv_ref, hn_ref, hns_ref):
    inv_n = 1.0 / N_NODES
    mean = s_ref[...] * inv_n
    var = q_ref[...] * inv_n - mean * mean
    scale = lax.rsqrt(var + 1e-5) * g_ref[...]
    hn = (h_ref[...] - mean) * scale + b_ref[...]
    hn_ref[...] = hn
    hns_ref[...] = hn * dinv_ref[...]


def _combine_body(s0_ref, s1_ref, hn_ref, hns_ref, h0_ref, dinv_ref, wc_ref,
                  h_ref, s_ref, q_ref):
    ax = (s0_ref[...] + s1_ref[...] + hns_ref[...]) * dinv_ref[...]
    t = (1.0 - ALPHA) * ax + ALPHA * h0_ref[...]
    u = jnp.dot(t, wc_ref[...], preferred_element_type=jnp.float32)
    h = jnp.maximum(u, 0.0) + hn_ref[...]
    h_ref[...] = h

    @pl.when(pl.program_id(0) == 0)
    def _():
        s_ref[...] = jnp.zeros_like(s_ref)
        q_ref[...] = jnp.zeros_like(q_ref)

    s_ref[...] += jnp.sum(h, axis=0, keepdims=True)
    q_ref[...] += jnp.sum(h * h, axis=0, keepdims=True)


def _final_body(h_ref, w_ref, b_ref, o_ref):
    logits = jnp.dot(h_ref[...], w_ref[...],
                     preferred_element_type=jnp.float32) + b_ref[...]
    m = jnp.max(logits, axis=-1, keepdims=True)
    s = logits - m
    lse = jnp.log(jnp.sum(jnp.exp(s), axis=-1, keepdims=True))
    o_ref[...] = s - lse


def _row_spec(cols):
    return pl.BlockSpec((BLK, cols), lambda i: (i, 0))


def _bcast_spec(rows, cols):
    return pl.BlockSpec((rows, cols), lambda i: (0, 0))


_STATS_OUT = [
    jax.ShapeDtypeStruct((1, F), jnp.float32),
    jax.ShapeDtypeStruct((1, F), jnp.float32),
]

_dinv = pl.pallas_call(
    _dinv_body,
    grid=(),
    out_shape=jax.ShapeDtypeStruct((ROWS_PAD // F, F), jnp.float32),
)

_prelude = pl.pallas_call(
    _prelude_body,
    grid=(NBLK,),
    in_specs=[_row_spec(F), _bcast_spec(F, F), _bcast_spec(1, F)],
    out_specs=[_row_spec(F), _bcast_spec(1, F), _bcast_spec(1, F)],
    out_shape=[jax.ShapeDtypeStruct((N_NODES, F), jnp.float32)] + _STATS_OUT,
)

_norm = pl.pallas_call(
    _norm_body,
    grid=(NBLK,),
    in_specs=[_row_spec(F), _bcast_spec(1, F), _bcast_spec(1, F),
              _bcast_spec(1, F), _bcast_spec(1, F), _row_spec(1)],
    out_specs=[_row_spec(F), _row_spec(F)],
    out_shape=[jax.ShapeDtypeStruct((N_NODES, F), jnp.float32),
               jax.ShapeDtypeStruct((N_NODES, F), jnp.float32)],
)

_combine = pl.pallas_call(
    _combine_body,
    grid=(NBLK,),
    in_specs=[_row_spec(F), _row_spec(F), _row_spec(F), _row_spec(F),
              _row_spec(F), _row_spec(1), _bcast_spec(F, F)],
    out_specs=[_row_spec(F), _bcast_spec(1, F), _bcast_spec(1, F)],
    out_shape=[jax.ShapeDtypeStruct((N_NODES, F), jnp.float32)] + _STATS_OUT,
)

_final = pl.pallas_call(
    _final_body,
    grid=(NBLK,),
    in_specs=[_row_spec(F), _bcast_spec(F, 40), _bcast_spec(1, 40)],
    out_specs=_row_spec(40),
    out_shape=jax.ShapeDtypeStruct((N_NODES, 40), jnp.float32),
)


def kernel(x, adj_t, W1, b1, gammas, bn_betas, Wc, W2, b2):
    pad = EP - E_EDGES
    pad_src = (jnp.arange(pad, dtype=jnp.int32) * 37) % N_NODES
    pad_dst = N_NODES + (jnp.arange(pad, dtype=jnp.int32) % (ROWS_PAD - N_NODES))
    src_p = jnp.concatenate([adj_t[0], pad_src]).reshape(NW, CPT, CH)
    dst_p = jnp.concatenate([adj_t[1], pad_dst.astype(jnp.int32)]).reshape(
        NW, CPT, CH)

    deg2 = _sc_degree(dst_p)
    dinv2d = _dinv(deg2[0].reshape(ROWS_PAD // F, F),
                   deg2[1].reshape(ROWS_PAD // F, F))
    dinv_col = dinv2d.reshape(ROWS_PAD, 1)[:N_NODES]

    h, ssum, ssq = _prelude(x, W1, b1.reshape(1, F))
    h0 = h
    for l in range(L_LAYERS):
        hn, hns = _norm(h, ssum, ssq, gammas[l].reshape(1, F),
                        bn_betas[l].reshape(1, F), dinv_col)
        s2 = _sc_propagate(hns, src_p, dst_p)
        h, ssum, ssq = _combine(s2[0, :N_NODES], s2[1, :N_NODES], hn, hns,
                                h0, dinv_col, Wc[l])
    return _final(h, W2, b2.reshape(1, 40))


# speedup vs baseline: 3.5290x; 1.1073x over previous
import functools

import jax
import jax.numpy as jnp
from jax import lax
from jax.experimental import pallas as pl
from jax.experimental.pallas import tpu as pltpu
from jax.experimental.pallas import tpu_sc as plsc

N_NODES = 10000
F = 128
E_EDGES = 320000
L_LAYERS = 4
ALPHA = 0.1

NC = 2
NS = 16
NW = NC * NS
CH = 128
CPT = 80
EPT = CPT * CH
EP = NW * EPT
GRP = 8
NIT = CPT // GRP
ROWS_PAD = 10240
RPT = ROWS_PAD // NS

BLK = 2000
NBLK = N_NODES // BLK

_mesh = plsc.VectorSubcoreMesh(core_axis_name="c", subcore_axis_name="s")


def _zero_fill(ref, nvec):
    z = jnp.zeros((16,), jnp.float32)

    def body(i, _):
        ref[i // 8, pl.ds((i % 8) * 16, 16)] = z
        return 0

    lax.fori_loop(0, nvec, body, 0)


def _sc_scalar_scatter(gather_src):
    scratch = [
        pltpu.VMEM_SHARED((ROWS_PAD,), jnp.float32),
        pltpu.VMEM((CH,), jnp.float32),
        pltpu.VMEM((RPT,), jnp.float32),
        pltpu.VMEM((CPT, CH), jnp.int32),
    ]
    if gather_src:
        scratch.append(pltpu.VMEM((CPT, CH), jnp.int32))
        scratch.append(pltpu.SemaphoreType.DMA)

    def body(*refs):
        if gather_src:
            (vals_hbm, src_hbm, dst_hbm, out_hbm, acc, val_v, zeros_v,
             dst_all, src_all, gsem) = refs
        else:
            dst_hbm, out_hbm, acc, val_v, zeros_v, dst_all = refs
        cid = lax.axis_index("c")
        sid = lax.axis_index("s")
        wid = cid * NS + sid

        pltpu.sync_copy(dst_hbm.at[wid], dst_all)
        if gather_src:
            pltpu.sync_copy(src_hbm.at[wid], src_all)
        else:
            one = jnp.ones((16,), jnp.float32)

            def fill(i, _):
                val_v[pl.ds(i * 16, 16)] = one
                return 0

            lax.fori_loop(0, CH // 16, fill, 0)

        zero = jnp.zeros((16,), jnp.float32)

        def zfill(i, _):
            zeros_v[pl.ds(i * 16, 16)] = zero
            return 0

        lax.fori_loop(0, RPT // 16, zfill, 0)
        rbase = sid * RPT
        pltpu.sync_copy(zeros_v, acc.at[pl.ds(rbase, RPT)])
        plsc.subcore_barrier()

        def chunk(i, _):
            if gather_src:
                pltpu.async_copy(vals_hbm.at[src_all.at[i]], val_v,
                                 gsem).wait()
            pltpu.sync_copy(val_v, acc.at[dst_all.at[i]], add=True)
            return 0

        lax.fori_loop(0, CPT, chunk, 0)
        plsc.subcore_barrier()
        pltpu.sync_copy(acc.at[pl.ds(rbase, RPT)],
                        out_hbm.at[cid, pl.ds(rbase, RPT)])

    return functools.partial(
        pl.kernel,
        out_type=jax.ShapeDtypeStruct((NC, ROWS_PAD), jnp.float32),
        mesh=_mesh,
        scratch_types=scratch,
    )(body)


_sc_degree = _sc_scalar_scatter(gather_src=False)
_sc_wdeg = _sc_scalar_scatter(gather_src=True)


@functools.partial(
    pl.kernel,
    out_type=jax.ShapeDtypeStruct((NC, ROWS_PAD, F), jnp.float32),
    mesh=_mesh,
    scratch_types=[
        pltpu.VMEM_SHARED((ROWS_PAD, F), jnp.float32),
        pltpu.VMEM((64, F), jnp.float32),
        pltpu.VMEM((4, CH), jnp.int32),
        pltpu.VMEM((4, CH), jnp.int32),
        pltpu.VMEM((4, CH), jnp.int32),
        pltpu.VMEM((4, CH), jnp.int32),
        pltpu.VMEM((CH, F), jnp.float32),
        pltpu.VMEM((CH, F), jnp.float32),
        pltpu.SemaphoreType.DMA,
        pltpu.SemaphoreType.DMA,
        pltpu.SemaphoreType.DMA,
        pltpu.SemaphoreType.DMA,
        pltpu.SemaphoreType.DMA,
        pltpu.SemaphoreType.DMA,
    ],
)
def _sc_propagate(hsc_hbm, src_hbm, dst_hbm, out_hbm, acc, zbuf, src0, dst0,
                  src1, dst1, rows_a, rows_b, gsa, gsb, ssa, ssb, is0, is1):
    cid = lax.axis_index("c")
    sid = lax.axis_index("s")
    wid = cid * NS + sid
    rbase = sid * RPT

    def load_idx(cbase, sbuf, dbuf, sem):
        pltpu.async_copy(src_hbm.at[wid, pl.ds(cbase, 4)], sbuf, sem)
        pltpu.async_copy(dst_hbm.at[wid, pl.ds(cbase, 4)], dbuf, sem)

    def wait_idx(sbuf, dbuf, sem):
        pltpu.make_async_copy(src_hbm.at[wid, pl.ds(0, 4)], sbuf, sem).wait()
        pltpu.make_async_copy(dst_hbm.at[wid, pl.ds(0, 4)], dbuf, sem).wait()

    def fire_g(sbuf, j, rbuf, sem):
        pltpu.async_copy(hsc_hbm.at[sbuf.at[j]], rbuf, sem)

    def wait_g(rbuf, sem):
        pltpu.make_async_copy(hsc_hbm.at[src0.at[0]], rbuf, sem).wait()

    def fire_s(rbuf, dbuf, j, sem):
        pltpu.async_copy(rbuf, acc.at[dbuf.at[j]], sem, add=True)

    def wait_s(rbuf, sem):
        pltpu.make_async_copy(rbuf, acc.at[dst0.at[0]], sem).wait()

    load_idx(0, src0, dst0, is0)
    _zero_fill(zbuf, 64 * 8)
    for r in range(0, RPT, 64):
        pltpu.sync_copy(zbuf, acc.at[pl.ds(rbase + r, 64)])
    wait_idx(src0, dst0, is0)
    fire_g(src0, 0, rows_a, gsa)
    fire_g(src0, 1, rows_b, gsb)
    plsc.subcore_barrier()

    def group(k, _):
        base = k * GRP
        load_idx(base + 4, src1, dst1, is1)
        wait_g(rows_a, gsa)
        fire_s(rows_a, dst0, 0, ssa)
        wait_s(rows_a, ssa)
        fire_g(src0, 2, rows_a, gsa)
        wait_g(rows_b, gsb)
        fire_s(rows_b, dst0, 1, ssb)
        wait_s(rows_b, ssb)
        fire_g(src0, 3, rows_b, gsb)
        wait_g(rows_a, gsa)
        fire_s(rows_a, dst0, 2, ssa)
        wait_s(rows_a, ssa)
        wait_idx(src1, dst1, is1)
        fire_g(src1, 0, rows_a, gsa)
        wait_g(rows_b, gsb)
        fire_s(rows_b, dst0, 3, ssb)
        wait_s(rows_b, ssb)
        fire_g(src1, 1, rows_b, gsb)

        @pl.when(k < NIT - 1)
        def _():
            load_idx(base + GRP, src0, dst0, is0)

        wait_g(rows_a, gsa)
        fire_s(rows_a, dst1, 0, ssa)
        wait_s(rows_a, ssa)
        fire_g(src1, 2, rows_a, gsa)
        wait_g(rows_b, gsb)
        fire_s(rows_b, dst1, 1, ssb)
        wait_s(rows_b, ssb)
        fire_g(src1, 3, rows_b, gsb)
        wait_g(rows_a, gsa)
        fire_s(rows_a, dst1, 2, ssa)
        wait_s(rows_a, ssa)

        @pl.when(k < NIT - 1)
        def _():
            wait_idx(src0, dst0, is0)
            fire_g(src0, 0, rows_a, gsa)

        wait_g(rows_b, gsb)
        fire_s(rows_b, dst1, 3, ssb)
        wait_s(rows_b, ssb)

        @pl.when(k < NIT - 1)
        def _():
            fire_g(src0, 1, rows_b, gsb)

        return 0

    lax.fori_loop(0, NIT, group, 0)
    plsc.subcore_barrier()
    pltpu.sync_copy(acc.at[pl.ds(rbase, RPT)],
                    out_hbm.at[cid, pl.ds(rbase, RPT)])


def _dinv_body(d0_ref, d1_ref, o_ref):
    deg = d0_ref[...] + d1_ref[...] + 1.0
    o_ref[...] = lax.rsqrt(deg)


def _wsum_body(w0_ref, w1_ref, o_ref):
    o_ref[...] = w0_ref[...] + w1_ref[...]


def _prelude_body(x_ref, w_ref, b_ref, dinv_ref, h_ref, hsc_ref, s_ref,
                  q_ref):
    h = jnp.dot(x_ref[...], w_ref[...], preferred_element_type=jnp.float32)
    h = jnp.maximum(h + b_ref[...], 0.0)
    h_ref[...] = h
    hsc_ref[...] = h * dinv_ref[...]

    @pl.when(pl.program_id(0) == 0)
    def _():
        s_ref[...] = jnp.zeros_like(s_ref)
        q_ref[...] = jnp.zeros_like(q_ref)

    s_ref[...] += jnp.sum(h, axis=0, keepdims=True)
    q_ref[...] += jnp.sum(h * h, axis=0, keepdims=True)


def _combine_body(s2a_ref, s2b_ref, h_ref, h0_ref, dinv_ref, w_ref, g_ref,
                  bet_ref, s_in_ref, q_in_ref, wc_ref, h_out_ref, hsc_ref,
                  s_ref, q_ref):
    inv_n = 1.0 / N_NODES
    m = s_in_ref[...] * inv_n
    var = q_in_ref[...] * inv_n - m * m
    a = lax.rsqrt(var + 1e-5) * g_ref[...]
    b = bet_ref[...] - m * a

    h = h_ref[...]
    dinv = dinv_ref[...]
    hn = h * a + b
    s_edges = (s2a_ref[0] + s2b_ref[0]) * a + b * w_ref[...]
    ax = dinv * (s_edges + dinv * hn)
    t = (1.0 - ALPHA) * ax + ALPHA * h0_ref[...]
    u = jnp.dot(t, wc_ref[...], preferred_element_type=jnp.float32)
    h_new = jnp.maximum(u, 0.0) + hn
    h_out_ref[...] = h_new
    hsc_ref[...] = h_new * dinv

    @pl.when(pl.program_id(0) == 0)
    def _():
        s_ref[...] = jnp.zeros_like(s_ref)
        q_ref[...] = jnp.zeros_like(q_ref)

    s_ref[...] += jnp.sum(h_new, axis=0, keepdims=True)
    q_ref[...] += jnp.sum(h_new * h_new, axis=0, keepdims=True)


def _final_body(h_ref, w_ref, b_ref, o_ref):
    logits = jnp.dot(h_ref[...], w_ref[...],
                     preferred_element_type=jnp.float32) + b_ref[...]
    m = jnp.max(logits, axis=-1, keepdims=True)
    s = logits - m
    lse = jnp.log(jnp.sum(jnp.exp(s), axis=-1, keepdims=True))
    o_ref[...] = s - lse


def _row_spec(cols):
    return pl.BlockSpec((BLK, cols), lambda i: (i, 0))


def _bcast_spec(rows, cols):
    return pl.BlockSpec((rows, cols), lambda i: (0, 0))


_STATS_OUT = [
    jax.ShapeDtypeStruct((1, F), jnp.float32),
    jax.ShapeDtypeStruct((1, F), jnp.float32),
]
_NF_OUT = jax.ShapeDtypeStruct((N_NODES, F), jnp.float32)

_dinv = pl.pallas_call(
    _dinv_body,
    grid=(),
    out_shape=jax.ShapeDtypeStruct((ROWS_PAD // F, F), jnp.float32),
)

_wsum = pl.pallas_call(
    _wsum_body,
    grid=(),
    out_shape=jax.ShapeDtypeStruct((ROWS_PAD // F, F), jnp.float32),
)

_prelude = pl.pallas_call(
    _prelude_body,
    grid=(NBLK,),
    in_specs=[_row_spec(F), _bcast_spec(F, F), _bcast_spec(1, F),
              _row_spec(1)],
    out_specs=[_row_spec(F), _row_spec(F), _bcast_spec(1, F),
               _bcast_spec(1, F)],
    out_shape=[_NF_OUT, _NF_OUT] + _STATS_OUT,
)

_combine = pl.pallas_call(
    _combine_body,
    grid=(NBLK,),
    in_specs=[
        pl.BlockSpec((1, BLK, F), lambda i: (0, i, 0)),
        pl.BlockSpec((1, BLK, F), lambda i: (1, i, 0)),
        _row_spec(F),
        _row_spec(F),
        _row_spec(1),
        _row_spec(1),
        _bcast_spec(1, F),
        _bcast_spec(1, F),
        _bcast_spec(1, F),
        _bcast_spec(1, F),
        _bcast_spec(F, F),
    ],
    out_specs=[_row_spec(F), _row_spec(F), _bcast_spec(1, F),
               _bcast_spec(1, F)],
    out_shape=[_NF_OUT, _NF_OUT] + _STATS_OUT,
)

_final = pl.pallas_call(
    _final_body,
    grid=(NBLK,),
    in_specs=[_row_spec(F), _bcast_spec(F, 40), _bcast_spec(1, 40)],
    out_specs=_row_spec(40),
    out_shape=jax.ShapeDtypeStruct((N_NODES, 40), jnp.float32),
)


def kernel(x, adj_t, W1, b1, gammas, bn_betas, Wc, W2, b2):
    pad = EP - E_EDGES
    pad_src = (jnp.arange(pad, dtype=jnp.int32) * 37) % N_NODES
    pad_dst = N_NODES + (jnp.arange(pad, dtype=jnp.int32) % (ROWS_PAD - N_NODES))
    src_p = jnp.concatenate([adj_t[0], pad_src]).reshape(NW, CPT, CH)
    dst_p = jnp.concatenate([adj_t[1], pad_dst.astype(jnp.int32)]).reshape(
        NW, CPT, CH)

    deg2 = _sc_degree(dst_p)
    dinv2d = _dinv(deg2[0].reshape(ROWS_PAD // F, F),
                   deg2[1].reshape(ROWS_PAD // F, F))
    dinv_col = dinv2d.reshape(ROWS_PAD, 1)[:N_NODES]
    w2 = _sc_wdeg(dinv2d.reshape(ROWS_PAD), src_p, dst_p)
    w2d = _wsum(w2[0].reshape(ROWS_PAD // F, F),
                w2[1].reshape(ROWS_PAD // F, F))
    w_col = w2d.reshape(ROWS_PAD, 1)[:N_NODES]

    h, hsc, ssum, ssq = _prelude(x, W1, b1.reshape(1, F), dinv_col)
    h0 = h
    for l in range(L_LAYERS):
        s2 = _sc_propagate(hsc, src_p, dst_p)
        h, hsc, ssum, ssq = _combine(s2, s2, h, h0, dinv_col, w_col,
                                     gammas[l].reshape(1, F),
                                     bn_betas[l].reshape(1, F), ssum, ssq,
                                     Wc[l])
    return _final(h, W2, b2.reshape(1, 40))


# speedup vs baseline: 3.9057x; 1.1067x over previous
import functools

import jax
import jax.numpy as jnp
from jax import lax
from jax.experimental import pallas as pl
from jax.experimental.pallas import tpu as pltpu
from jax.experimental.pallas import tpu_sc as plsc

N_NODES = 10000
F = 128
E_EDGES = 320000
L_LAYERS = 4
ALPHA = 0.1

NC = 2
NS = 16
NW = NC * NS
CH = 128
CPT = 80
EPT = CPT * CH
EP = NW * EPT
GRP = 8
NIT = CPT // GRP
ROWS_PAD = 10240
RPT = ROWS_PAD // NS

BLK = 2000
NBLK = N_NODES // BLK

_mesh = plsc.VectorSubcoreMesh(core_axis_name="c", subcore_axis_name="s")


def _zero_fill(ref, nvec):
    z = jnp.zeros((16,), jnp.float32)

    def body(i, _):
        ref[i // 8, pl.ds((i % 8) * 16, 16)] = z
        return 0

    lax.fori_loop(0, nvec, body, 0)


def _sc_scalar_scatter(gather_src):
    scratch = [
        pltpu.VMEM_SHARED((ROWS_PAD,), jnp.float32),
        pltpu.VMEM((CH,), jnp.float32),
        pltpu.VMEM((RPT,), jnp.float32),
        pltpu.VMEM((CPT, CH), jnp.int32),
    ]
    if gather_src:
        scratch.append(pltpu.VMEM((CPT, CH), jnp.int32))
        scratch.append(pltpu.VMEM((ROWS_PAD // F, F), jnp.float32))

    def body(*refs):
        if gather_src:
            (vals_hbm, src_hbm, dst_hbm, out_hbm, acc, val_v, zeros_v,
             dst_all, src_all, vals_v) = refs
        else:
            dst_hbm, out_hbm, acc, val_v, zeros_v, dst_all = refs
        cid = lax.axis_index("c")
        sid = lax.axis_index("s")
        wid = cid * NS + sid

        pltpu.sync_copy(dst_hbm.at[wid], dst_all)
        if gather_src:
            pltpu.sync_copy(src_hbm.at[wid], src_all)
            pltpu.sync_copy(vals_hbm, vals_v)
        else:
            one = jnp.ones((16,), jnp.float32)

            def fill(i, _):
                val_v[pl.ds(i * 16, 16)] = one
                return 0

            lax.fori_loop(0, CH // 16, fill, 0)

        zero = jnp.zeros((16,), jnp.float32)

        def zfill(i, _):
            zeros_v[pl.ds(i * 16, 16)] = zero
            return 0

        lax.fori_loop(0, RPT // 16, zfill, 0)
        rbase = sid * RPT
        pltpu.sync_copy(zeros_v, acc.at[pl.ds(rbase, RPT)])
        plsc.subcore_barrier()

        def chunk(i, _):
            if gather_src:
                for g in range(CH // 16):
                    s16 = src_all[i, pl.ds(g * 16, 16)]
                    val_v[pl.ds(g * 16, 16)] = plsc.load_gather(
                        vals_v, [s16 >> 7, s16 & 127])
            pltpu.sync_copy(val_v, acc.at[dst_all.at[i]], add=True)
            return 0

        lax.fori_loop(0, CPT, chunk, 0)
        plsc.subcore_barrier()
        pltpu.sync_copy(acc.at[pl.ds(rbase, RPT)],
                        out_hbm.at[cid, pl.ds(rbase, RPT)])

    return functools.partial(
        pl.kernel,
        out_type=jax.ShapeDtypeStruct((NC, ROWS_PAD), jnp.float32),
        mesh=_mesh,
        scratch_types=scratch,
        compiler_params=pltpu.CompilerParams(
            needs_layout_passes=not gather_src),
    )(body)


_sc_degree = _sc_scalar_scatter(gather_src=False)
_sc_wdeg = _sc_scalar_scatter(gather_src=True)


@functools.partial(
    pl.kernel,
    out_type=jax.ShapeDtypeStruct((NC, ROWS_PAD, F), jnp.float32),
    mesh=_mesh,
    scratch_types=[
        pltpu.VMEM_SHARED((ROWS_PAD, F), jnp.float32),
        pltpu.VMEM((64, F), jnp.float32),
        pltpu.VMEM((4, CH), jnp.int32),
        pltpu.VMEM((4, CH), jnp.int32),
        pltpu.VMEM((4, CH), jnp.int32),
        pltpu.VMEM((4, CH), jnp.int32),
        pltpu.VMEM((CH, F), jnp.float32),
        pltpu.VMEM((CH, F), jnp.float32),
        pltpu.SemaphoreType.DMA,
        pltpu.SemaphoreType.DMA,
        pltpu.SemaphoreType.DMA,
        pltpu.SemaphoreType.DMA,
        pltpu.SemaphoreType.DMA,
        pltpu.SemaphoreType.DMA,
    ],
)
def _sc_propagate(hsc_hbm, src_hbm, dst_hbm, out_hbm, acc, zbuf, src0, dst0,
                  src1, dst1, rows_a, rows_b, gsa, gsb, ssa, ssb, is0, is1):
    cid = lax.axis_index("c")
    sid = lax.axis_index("s")
    wid = cid * NS + sid
    rbase = sid * RPT

    def load_idx(cbase, sbuf, dbuf, sem):
        pltpu.async_copy(src_hbm.at[wid, pl.ds(cbase, 4)], sbuf, sem)
        pltpu.async_copy(dst_hbm.at[wid, pl.ds(cbase, 4)], dbuf, sem)

    def wait_idx(sbuf, dbuf, sem):
        pltpu.make_async_copy(src_hbm.at[wid, pl.ds(0, 4)], sbuf, sem).wait()
        pltpu.make_async_copy(dst_hbm.at[wid, pl.ds(0, 4)], dbuf, sem).wait()

    def fire_g(sbuf, j, rbuf, sem):
        pltpu.async_copy(hsc_hbm.at[sbuf.at[j]], rbuf, sem)

    def wait_g(rbuf, sem):
        pltpu.make_async_copy(hsc_hbm.at[src0.at[0]], rbuf, sem).wait()

    def fire_s(rbuf, dbuf, j, sem):
        pltpu.async_copy(rbuf, acc.at[dbuf.at[j]], sem, add=True)

    def wait_s(rbuf, sem):
        pltpu.make_async_copy(rbuf, acc.at[dst0.at[0]], sem).wait()

    load_idx(0, src0, dst0, is0)
    _zero_fill(zbuf, 64 * 8)
    for r in range(0, RPT, 64):
        pltpu.sync_copy(zbuf, acc.at[pl.ds(rbase + r, 64)])
    wait_idx(src0, dst0, is0)
    fire_g(src0, 0, rows_a, gsa)
    fire_g(src0, 1, rows_b, gsb)
    plsc.subcore_barrier()

    def group(k, _):
        base = k * GRP
        load_idx(base + 4, src1, dst1, is1)
        wait_g(rows_a, gsa)
        fire_s(rows_a, dst0, 0, ssa)
        wait_s(rows_a, ssa)
        fire_g(src0, 2, rows_a, gsa)
        wait_g(rows_b, gsb)
        fire_s(rows_b, dst0, 1, ssb)
        wait_s(rows_b, ssb)
        fire_g(src0, 3, rows_b, gsb)
        wait_g(rows_a, gsa)
        fire_s(rows_a, dst0, 2, ssa)
        wait_s(rows_a, ssa)
        wait_idx(src1, dst1, is1)
        fire_g(src1, 0, rows_a, gsa)
        wait_g(rows_b, gsb)
        fire_s(rows_b, dst0, 3, ssb)
        wait_s(rows_b, ssb)
        fire_g(src1, 1, rows_b, gsb)

        @pl.when(k < NIT - 1)
        def _():
            load_idx(base + GRP, src0, dst0, is0)

        wait_g(rows_a, gsa)
        fire_s(rows_a, dst1, 0, ssa)
        wait_s(rows_a, ssa)
        fire_g(src1, 2, rows_a, gsa)
        wait_g(rows_b, gsb)
        fire_s(rows_b, dst1, 1, ssb)
        wait_s(rows_b, ssb)
        fire_g(src1, 3, rows_b, gsb)
        wait_g(rows_a, gsa)
        fire_s(rows_a, dst1, 2, ssa)
        wait_s(rows_a, ssa)

        @pl.when(k < NIT - 1)
        def _():
            wait_idx(src0, dst0, is0)
            fire_g(src0, 0, rows_a, gsa)

        wait_g(rows_b, gsb)
        fire_s(rows_b, dst1, 3, ssb)
        wait_s(rows_b, ssb)

        @pl.when(k < NIT - 1)
        def _():
            fire_g(src0, 1, rows_b, gsb)

        return 0

    lax.fori_loop(0, NIT, group, 0)
    plsc.subcore_barrier()
    pltpu.sync_copy(acc.at[pl.ds(rbase, RPT)],
                    out_hbm.at[cid, pl.ds(rbase, RPT)])


def _dinv_body(d0_ref, d1_ref, o_ref):
    deg = d0_ref[...] + d1_ref[...] + 1.0
    o_ref[...] = lax.rsqrt(deg)


def _wsum_body(w0_ref, w1_ref, o_ref):
    o_ref[...] = w0_ref[...] + w1_ref[...]


def _prelude_body(x_ref, w_ref, b_ref, dinv_ref, h_ref, hsc_ref, s_ref,
                  q_ref):
    h = jnp.dot(x_ref[...], w_ref[...], preferred_element_type=jnp.float32)
    h = jnp.maximum(h + b_ref[...], 0.0)
    h_ref[...] = h
    hsc_ref[...] = h * dinv_ref[...]

    @pl.when(pl.program_id(0) == 0)
    def _():
        s_ref[...] = jnp.zeros_like(s_ref)
        q_ref[...] = jnp.zeros_like(q_ref)

    s_ref[...] += jnp.sum(h, axis=0, keepdims=True)
    q_ref[...] += jnp.sum(h * h, axis=0, keepdims=True)


def _combine_body(s2a_ref, s2b_ref, h_ref, h0_ref, dinv_ref, w_ref, g_ref,
                  bet_ref, s_in_ref, q_in_ref, wc_ref, h_out_ref, hsc_ref,
                  s_ref, q_ref):
    inv_n = 1.0 / N_NODES
    m = s_in_ref[...] * inv_n
    var = q_in_ref[...] * inv_n - m * m
    a = lax.rsqrt(var + 1e-5) * g_ref[...]
    b = bet_ref[...] - m * a

    h = h_ref[...]
    dinv = dinv_ref[...]
    hn = h * a + b
    s_edges = (s2a_ref[0] + s2b_ref[0]) * a + b * w_ref[...]
    ax = dinv * (s_edges + dinv * hn)
    t = (1.0 - ALPHA) * ax + ALPHA * h0_ref[...]
    u = jnp.dot(t, wc_ref[...], preferred_element_type=jnp.float32)
    h_new = jnp.maximum(u, 0.0) + hn
    h_out_ref[...] = h_new
    hsc_ref[...] = h_new * dinv

    @pl.when(pl.program_id(0) == 0)
    def _():
        s_ref[...] = jnp.zeros_like(s_ref)
        q_ref[...] = jnp.zeros_like(q_ref)

    s_ref[...] += jnp.sum(h_new, axis=0, keepdims=True)
    q_ref[...] += jnp.sum(h_new * h_new, axis=0, keepdims=True)


def _final_body(h_ref, w_ref, b_ref, o_ref):
    logits = jnp.dot(h_ref[...], w_ref[...],
                     preferred_element_type=jnp.float32) + b_ref[...]
    m = jnp.max(logits, axis=-1, keepdims=True)
    s = logits - m
    lse = jnp.log(jnp.sum(jnp.exp(s), axis=-1, keepdims=True))
    o_ref[...] = s - lse


def _row_spec(cols):
    return pl.BlockSpec((BLK, cols), lambda i: (i, 0))


def _bcast_spec(rows, cols):
    return pl.BlockSpec((rows, cols), lambda i: (0, 0))


_STATS_OUT = [
    jax.ShapeDtypeStruct((1, F), jnp.float32),
    jax.ShapeDtypeStruct((1, F), jnp.float32),
]
_NF_OUT = jax.ShapeDtypeStruct((N_NODES, F), jnp.float32)

_dinv = pl.pallas_call(
    _dinv_body,
    grid=(),
    out_shape=jax.ShapeDtypeStruct((ROWS_PAD // F, F), jnp.float32),
)

_wsum = pl.pallas_call(
    _wsum_body,
    grid=(),
    out_shape=jax.ShapeDtypeStruct((ROWS_PAD // F, F), jnp.float32),
)

_prelude = pl.pallas_call(
    _prelude_body,
    grid=(NBLK,),
    in_specs=[_row_spec(F), _bcast_spec(F, F), _bcast_spec(1, F),
              _row_spec(1)],
    out_specs=[_row_spec(F), _row_spec(F), _bcast_spec(1, F),
               _bcast_spec(1, F)],
    out_shape=[_NF_OUT, _NF_OUT] + _STATS_OUT,
)

_combine = pl.pallas_call(
    _combine_body,
    grid=(NBLK,),
    in_specs=[
        pl.BlockSpec((1, BLK, F), lambda i: (0, i, 0)),
        pl.BlockSpec((1, BLK, F), lambda i: (1, i, 0)),
        _row_spec(F),
        _row_spec(F),
        _row_spec(1),
        _row_spec(1),
        _bcast_spec(1, F),
        _bcast_spec(1, F),
        _bcast_spec(1, F),
        _bcast_spec(1, F),
        _bcast_spec(F, F),
    ],
    out_specs=[_row_spec(F), _row_spec(F), _bcast_spec(1, F),
               _bcast_spec(1, F)],
    out_shape=[_NF_OUT, _NF_OUT] + _STATS_OUT,
)

_final = pl.pallas_call(
    _final_body,
    grid=(NBLK,),
    in_specs=[_row_spec(F), _bcast_spec(F, 40), _bcast_spec(1, 40)],
    out_specs=_row_spec(40),
    out_shape=jax.ShapeDtypeStruct((N_NODES, 40), jnp.float32),
)


def kernel(x, adj_t, W1, b1, gammas, bn_betas, Wc, W2, b2):
    pad = EP - E_EDGES
    pad_src = (jnp.arange(pad, dtype=jnp.int32) * 37) % N_NODES
    pad_dst = N_NODES + (jnp.arange(pad, dtype=jnp.int32) % (ROWS_PAD - N_NODES))
    src_p = jnp.concatenate([adj_t[0], pad_src]).reshape(NW, CPT, CH)
    dst_p = jnp.concatenate([adj_t[1], pad_dst.astype(jnp.int32)]).reshape(
        NW, CPT, CH)

    deg2 = _sc_degree(dst_p)
    dinv2d = _dinv(deg2[0].reshape(ROWS_PAD // F, F),
                   deg2[1].reshape(ROWS_PAD // F, F))
    dinv_col = dinv2d.reshape(ROWS_PAD, 1)[:N_NODES]
    w2 = _sc_wdeg(dinv2d, src_p, dst_p)
    w2d = _wsum(w2[0].reshape(ROWS_PAD // F, F),
                w2[1].reshape(ROWS_PAD // F, F))
    w_col = w2d.reshape(ROWS_PAD, 1)[:N_NODES]

    h, hsc, ssum, ssq = _prelude(x, W1, b1.reshape(1, F), dinv_col)
    h0 = h
    for l in range(L_LAYERS):
        s2 = _sc_propagate(hsc, src_p, dst_p)
        h, hsc, ssum, ssq = _combine(s2, s2, h, h0, dinv_col, w_col,
                                     gammas[l].reshape(1, F),
                                     bn_betas[l].reshape(1, F), ssum, ssq,
                                     Wc[l])
    return _final(h, W2, b2.reshape(1, 40))
